# baseline (speedup 1.0000x reference)
import functools

import jax
import jax.numpy as jnp
from jax import lax
from jax.experimental import pallas as pl
from jax.experimental.pallas import tpu as pltpu
from jax.experimental.pallas import tpu_sc as plsc

N = 10000
E = 320000
D = 128
EPS = 1e-5

NC = 2
NS = 16
CHUNK = 128
NCHUNKS = E // CHUNK
SUB_ROWS = 624
TAIL = N - NS * SUB_ROWS
ZBLK = 104

BLK = 2000


def _ln(v, g, b):
    m = jnp.mean(v, axis=-1, keepdims=True)
    c = v - m
    s = jnp.mean(c * c, axis=-1, keepdims=True)
    return c * lax.rsqrt(s + EPS) * g + b


def _ffn_body(x_ref, w1_ref, b1_ref, w2_ref, b2_ref, g_ref, b_ref, o_ref):
    x = x_ref[...]
    h = jnp.maximum(
        jnp.dot(x, w1_ref[...], preferred_element_type=jnp.float32)
        + b1_ref[...], 0.0)
    f = jnp.dot(h, w2_ref[...], preferred_element_type=jnp.float32) + b2_ref[...]
    o_ref[...] = _ln(f + x, g_ref[...], b_ref[...])


def _ffn_ln(xa, w1, b1, w2, b2, g, b):
    row = pl.BlockSpec((BLK, D), lambda i: (i, 0))
    mat = pl.BlockSpec((D, D), lambda i: (0, 0))
    vec = pl.BlockSpec((1, D), lambda i: (0, 0))
    return pl.pallas_call(
        _ffn_body,
        grid=(N // BLK,),
        in_specs=[row, mat, vec, mat, vec, vec, vec],
        out_specs=row,
        out_shape=jax.ShapeDtypeStruct((N, D), jnp.float32),
    )(xa, w1, b1.reshape(1, D), w2, b2.reshape(1, D),
      g.reshape(1, D), b.reshape(1, D))


def _sc_body(ff_hbm, src_hbm, dst_hbm, out_hbm,
             acc, rows, sidx, didx, sem):
    c = lax.axis_index("c")
    s = lax.axis_index("s")

    @pl.loop(0, CHUNK)
    def _(r):
        @pl.loop(0, D, step=16)
        def _(k):
            rows[r, pl.ds(k, 16)] = jnp.zeros((16,), jnp.float32)

    base = s * SUB_ROWS

    @pl.loop(0, SUB_ROWS // ZBLK)
    def _(k):
        pltpu.sync_copy(rows.at[pl.ds(0, ZBLK)],
                        acc.at[pl.ds(base + k * ZBLK, ZBLK)])

    @pl.when(s == NS - 1)
    def _():
        pltpu.sync_copy(rows.at[pl.ds(0, TAIL)],
                        acc.at[pl.ds(NS * SUB_ROWS, TAIL)])

    plsc.subcore_barrier()

    w = c * NS + s

    @pl.loop(w, NCHUNKS, step=NC * NS)
    def _(i):
        pltpu.sync_copy(src_hbm.at[pl.ds(i, 1)], sidx)
        pltpu.sync_copy(dst_hbm.at[pl.ds(i, 1)], didx)
        pltpu.async_copy(ff_hbm.at[sidx.at[0]], rows, sem).wait()
        pltpu.sync_copy(rows, acc.at[didx.at[0]], add=True)

    plsc.subcore_barrier()

    obase = c * N + s * SUB_ROWS

    @pl.loop(0, SUB_ROWS // ZBLK)
    def _(k):
        pltpu.sync_copy(acc.at[pl.ds(base + k * ZBLK, ZBLK)],
                        out_hbm.at[pl.ds(obase + k * ZBLK, ZBLK)])

    @pl.when(s == NS - 1)
    def _():
        pltpu.sync_copy(acc.at[pl.ds(NS * SUB_ROWS, TAIL)],
                        out_hbm.at[pl.ds(c * N + NS * SUB_ROWS, TAIL)])


@functools.cache
def _get_sc_seg():
  return pl.kernel(
    _sc_body,
    out_type=jax.ShapeDtypeStruct((NC * N, D), jnp.float32),
    mesh=plsc.VectorSubcoreMesh(core_axis_name="c", subcore_axis_name="s",
                                num_cores=NC, num_subcores=NS),
    scratch_types=[
        pltpu.VMEM_SHARED((N, D), jnp.float32),
        pltpu.VMEM((CHUNK, D), jnp.float32),
        pltpu.VMEM((1, CHUNK), jnp.int32),
        pltpu.VMEM((1, CHUNK), jnp.int32),
        pltpu.SemaphoreType.DMA,
    ],
  )


def _agg_body(p_ref, deg_ref, ff_ref, wg_ref, bg_ref, g_ref, b_ref, o_ref):
    summ = p_ref[0] + p_ref[1]
    deg = deg_ref[0, :, 0:1] + deg_ref[1, :, 0:1]
    agg = summ / jnp.maximum(deg, 1.0)
    h = jnp.maximum(
        jnp.dot(agg, wg_ref[...], preferred_element_type=jnp.float32)
        + bg_ref[...], 0.0)
    o_ref[...] = _ln(h + ff_ref[...], g_ref[...], b_ref[...])


def _agg_ln(part, deg, ff, wg, bg, g, b):
    return pl.pallas_call(
        _agg_body,
        grid=(N // BLK,),
        in_specs=[
            pl.BlockSpec((2, BLK, D), lambda i: (0, i, 0)),
            pl.BlockSpec((2, BLK, D), lambda i: (0, i, 0)),
            pl.BlockSpec((BLK, D), lambda i: (i, 0)),
            pl.BlockSpec((D, D), lambda i: (0, 0)),
            pl.BlockSpec((1, D), lambda i: (0, 0)),
            pl.BlockSpec((1, D), lambda i: (0, 0)),
            pl.BlockSpec((1, D), lambda i: (0, 0)),
        ],
        out_specs=pl.BlockSpec((BLK, D), lambda i: (i, 0)),
        out_shape=jax.ShapeDtypeStruct((N, D), jnp.float32),
    )(part.reshape(NC, N, D), deg.reshape(NC, N, D), ff, wg,
      bg.reshape(1, D), g.reshape(1, D), b.reshape(1, D))


def kernel(x, edge_index,
           Wff1_0, bff1_0, Wff2_0, bff2_0, gff_0, boff_0, Wg_0, bg_0, gln_0, bln_0,
           Wff1_1, bff1_1, Wff2_1, bff2_1, gff_1, boff_1, Wg_1, bg_1, gln_1, bln_1):
    src = edge_index[0].reshape(NCHUNKS, CHUNK)
    dst = edge_index[1].reshape(NCHUNKS, CHUNK)
    layers = [
        (Wff1_0, bff1_0, Wff2_0, bff2_0, gff_0, boff_0, Wg_0, bg_0, gln_0, bln_0),
        (Wff1_1, bff1_1, Wff2_1, bff2_1, gff_1, boff_1, Wg_1, bg_1, gln_1, bln_1),
    ]
    ones_row = jnp.ones((8, D), jnp.float32)
    zidx = jnp.zeros((NCHUNKS, CHUNK), jnp.int32)
    deg = _get_sc_seg()(ones_row, zidx, dst)
    out = x
    for (w1, b1, w2, b2, gff, boff, wg, bg, gln, bln) in layers:
        ff = _ffn_ln(out, w1, b1, w2, b2, gff, boff)
        part = _get_sc_seg()(ff, src, dst)
        out = _agg_ln(part, deg, ff, wg, bg, gln, bln)
    return out


# speedup vs baseline: 19.2817x; 19.2817x over previous
import functools

import jax
import jax.numpy as jnp
from jax import lax
from jax.experimental import pallas as pl
from jax.experimental.pallas import tpu as pltpu
from jax.experimental.pallas import tpu_sc as plsc

N = 10000
E = 320000
D = 128
EPS = 1e-5

NC = 2
NS = 16
CHUNK = 128
NCHUNKS = E // CHUNK
SUB_ROWS = 624
TAIL = N - NS * SUB_ROWS
ZBLK = 104

BLK = 2000


def _ln(v, g, b):
    m = jnp.mean(v, axis=-1, keepdims=True)
    c = v - m
    s = jnp.mean(c * c, axis=-1, keepdims=True)
    return c * lax.rsqrt(s + EPS) * g + b


def _ffn_body(x_ref, w1_ref, b1_ref, w2_ref, b2_ref, g_ref, b_ref, o_ref):
    x = x_ref[...]
    h = jnp.maximum(
        jnp.dot(x, w1_ref[...], preferred_element_type=jnp.float32)
        + b1_ref[...], 0.0)
    f = jnp.dot(h, w2_ref[...], preferred_element_type=jnp.float32) + b2_ref[...]
    o_ref[...] = _ln(f + x, g_ref[...], b_ref[...])


def _ffn_ln(xa, w1, b1, w2, b2, g, b):
    row = pl.BlockSpec((BLK, D), lambda i: (i, 0))
    mat = pl.BlockSpec((D, D), lambda i: (0, 0))
    vec = pl.BlockSpec((1, D), lambda i: (0, 0))
    return pl.pallas_call(
        _ffn_body,
        grid=(N // BLK,),
        in_specs=[row, mat, vec, mat, vec, vec, vec],
        out_specs=row,
        out_shape=jax.ShapeDtypeStruct((N, D), jnp.float32),
    )(xa, w1, b1.reshape(1, D), w2, b2.reshape(1, D),
      g.reshape(1, D), b.reshape(1, D))


def _sc_deg_body(dst_hbm, deg_hbm, acc, rows, didx, sem):
    c = lax.axis_index("c")
    s = lax.axis_index("s")

    @pl.loop(0, CHUNK)
    def _(r):
        @pl.loop(0, D, step=16)
        def _(k):
            rows[r, pl.ds(k, 16)] = jnp.zeros((16,), jnp.float32)

    base = s * SUB_ROWS

    @pl.loop(0, SUB_ROWS // ZBLK)
    def _(k):
        pltpu.sync_copy(rows.at[pl.ds(0, ZBLK)],
                        acc.at[pl.ds(base + k * ZBLK, ZBLK)])

    @pl.when(s == NS - 1)
    def _():
        pltpu.sync_copy(rows.at[pl.ds(0, TAIL)],
                        acc.at[pl.ds(NS * SUB_ROWS, TAIL)])

    @pl.loop(0, CHUNK)
    def _(r):
        @pl.loop(0, D, step=16)
        def _(k):
            rows[r, pl.ds(k, 16)] = jnp.ones((16,), jnp.float32)

    plsc.subcore_barrier()

    w = c * NS + s

    @pl.loop(w, NCHUNKS, step=NC * NS)
    def _(i):
        pltpu.sync_copy(dst_hbm.at[pl.ds(i, 1)], didx)
        pltpu.sync_copy(rows, acc.at[didx.at[0]], add=True)

    plsc.subcore_barrier()

    obase = c * N + s * SUB_ROWS

    @pl.loop(0, SUB_ROWS // ZBLK)
    def _(k):
        pltpu.sync_copy(acc.at[pl.ds(base + k * ZBLK, ZBLK)],
                        deg_hbm.at[pl.ds(obase + k * ZBLK, ZBLK)])

    @pl.when(s == NS - 1)
    def _():
        pltpu.sync_copy(acc.at[pl.ds(NS * SUB_ROWS, TAIL)],
                        deg_hbm.at[pl.ds(c * N + NS * SUB_ROWS, TAIL)])


@functools.cache
def _get_sc_deg():
  return pl.kernel(
    _sc_deg_body,
    out_type=jax.ShapeDtypeStruct((NC * N, D), jnp.float32),
    mesh=plsc.VectorSubcoreMesh(core_axis_name="c", subcore_axis_name="s",
                                num_cores=NC, num_subcores=NS),
    scratch_types=[
        pltpu.VMEM_SHARED((N, D), jnp.float32),
        pltpu.VMEM((CHUNK, D), jnp.float32),
        pltpu.VMEM((1, CHUNK), jnp.int32),
        pltpu.SemaphoreType.DMA,
    ],
  )


def _sc_body(ff_hbm, src_hbm, dst_hbm, out_hbm,
             acc, rows, sidx, didx, sem):
    c = lax.axis_index("c")
    s = lax.axis_index("s")

    @pl.loop(0, CHUNK)
    def _(r):
        @pl.loop(0, D, step=16)
        def _(k):
            rows[r, pl.ds(k, 16)] = jnp.zeros((16,), jnp.float32)

    base = s * SUB_ROWS

    @pl.loop(0, SUB_ROWS // ZBLK)
    def _(k):
        pltpu.sync_copy(rows.at[pl.ds(0, ZBLK)],
                        acc.at[pl.ds(base + k * ZBLK, ZBLK)])

    @pl.when(s == NS - 1)
    def _():
        pltpu.sync_copy(rows.at[pl.ds(0, TAIL)],
                        acc.at[pl.ds(NS * SUB_ROWS, TAIL)])

    plsc.subcore_barrier()

    w = c * NS + s

    @pl.loop(w, NCHUNKS, step=NC * NS)
    def _(i):
        pltpu.sync_copy(src_hbm.at[pl.ds(i, 1)], sidx)
        pltpu.sync_copy(dst_hbm.at[pl.ds(i, 1)], didx)
        pltpu.async_copy(ff_hbm.at[sidx.at[0]], rows, sem).wait()
        pltpu.sync_copy(rows, acc.at[didx.at[0]], add=True)

    plsc.subcore_barrier()

    obase = c * N + s * SUB_ROWS

    @pl.loop(0, SUB_ROWS // ZBLK)
    def _(k):
        pltpu.sync_copy(acc.at[pl.ds(base + k * ZBLK, ZBLK)],
                        out_hbm.at[pl.ds(obase + k * ZBLK, ZBLK)])

    @pl.when(s == NS - 1)
    def _():
        pltpu.sync_copy(acc.at[pl.ds(NS * SUB_ROWS, TAIL)],
                        out_hbm.at[pl.ds(c * N + NS * SUB_ROWS, TAIL)])


@functools.cache
def _get_sc_seg():
  return pl.kernel(
    _sc_body,
    out_type=jax.ShapeDtypeStruct((NC * N, D), jnp.float32),
    mesh=plsc.VectorSubcoreMesh(core_axis_name="c", subcore_axis_name="s",
                                num_cores=NC, num_subcores=NS),
    scratch_types=[
        pltpu.VMEM_SHARED((N, D), jnp.float32),
        pltpu.VMEM((CHUNK, D), jnp.float32),
        pltpu.VMEM((1, CHUNK), jnp.int32),
        pltpu.VMEM((1, CHUNK), jnp.int32),
        pltpu.SemaphoreType.DMA,
    ],
  )


def _agg_body(p_ref, deg_ref, ff_ref, wg_ref, bg_ref, g_ref, b_ref, o_ref):
    summ = p_ref[0] + p_ref[1]
    deg = deg_ref[0, :, 0:1] + deg_ref[1, :, 0:1]
    agg = summ / jnp.maximum(deg, 1.0)
    h = jnp.maximum(
        jnp.dot(agg, wg_ref[...], preferred_element_type=jnp.float32)
        + bg_ref[...], 0.0)
    o_ref[...] = _ln(h + ff_ref[...], g_ref[...], b_ref[...])


def _agg_ln(part, deg, ff, wg, bg, g, b):
    return pl.pallas_call(
        _agg_body,
        grid=(N // BLK,),
        in_specs=[
            pl.BlockSpec((2, BLK, D), lambda i: (0, i, 0)),
            pl.BlockSpec((2, BLK, D), lambda i: (0, i, 0)),
            pl.BlockSpec((BLK, D), lambda i: (i, 0)),
            pl.BlockSpec((D, D), lambda i: (0, 0)),
            pl.BlockSpec((1, D), lambda i: (0, 0)),
            pl.BlockSpec((1, D), lambda i: (0, 0)),
            pl.BlockSpec((1, D), lambda i: (0, 0)),
        ],
        out_specs=pl.BlockSpec((BLK, D), lambda i: (i, 0)),
        out_shape=jax.ShapeDtypeStruct((N, D), jnp.float32),
    )(part.reshape(NC, N, D), deg.reshape(NC, N, D), ff, wg,
      bg.reshape(1, D), g.reshape(1, D), b.reshape(1, D))


def kernel(x, edge_index,
           Wff1_0, bff1_0, Wff2_0, bff2_0, gff_0, boff_0, Wg_0, bg_0, gln_0, bln_0,
           Wff1_1, bff1_1, Wff2_1, bff2_1, gff_1, boff_1, Wg_1, bg_1, gln_1, bln_1):
    src = edge_index[0].reshape(NCHUNKS, CHUNK)
    dst = edge_index[1].reshape(NCHUNKS, CHUNK)
    layers = [
        (Wff1_0, bff1_0, Wff2_0, bff2_0, gff_0, boff_0, Wg_0, bg_0, gln_0, bln_0),
        (Wff1_1, bff1_1, Wff2_1, bff2_1, gff_1, boff_1, Wg_1, bg_1, gln_1, bln_1),
    ]
    deg = _get_sc_deg()(dst)
    out = x
    for (w1, b1, w2, b2, gff, boff, wg, bg, gln, bln) in layers:
        ff = _ffn_ln(out, w1, b1, w2, b2, gff, boff)
        part = _get_sc_seg()(ff, src, dst)
        out = _agg_ln(part, deg, ff, wg, bg, gln, bln)
    return out


# speedup vs baseline: 31.5282x; 1.6351x over previous
import functools

import jax
import jax.numpy as jnp
from jax import lax
from jax.experimental import pallas as pl
from jax.experimental.pallas import tpu as pltpu
from jax.experimental.pallas import tpu_sc as plsc

N = 10000
E = 320000
D = 128
EPS = 1e-5

NC = 2
NS = 16
CHUNK = 128
NCHUNKS = E // CHUNK
SUB_ROWS = 624
TAIL = N - NS * SUB_ROWS
ZBLK = 104
WCH = 80
NCHP = NC * NS * WCH
NA = N + CHUNK
HW = WCH // 2

BLK = 2000


def _ln(v, g, b):
    m = jnp.mean(v, axis=-1, keepdims=True)
    c = v - m
    s = jnp.mean(c * c, axis=-1, keepdims=True)
    return c * lax.rsqrt(s + EPS) * g + b


def _ffn_body(x_ref, w1_ref, b1_ref, w2_ref, b2_ref, g_ref, b_ref, o_ref):
    x = x_ref[...]
    h = jnp.maximum(
        jnp.dot(x, w1_ref[...], preferred_element_type=jnp.float32)
        + b1_ref[...], 0.0)
    f = jnp.dot(h, w2_ref[...], preferred_element_type=jnp.float32) + b2_ref[...]
    o_ref[...] = _ln(f + x, g_ref[...], b_ref[...])


def _ffn_ln(xa, w1, b1, w2, b2, g, b):
    row = pl.BlockSpec((BLK, D), lambda i: (i, 0))
    mat = pl.BlockSpec((D, D), lambda i: (0, 0))
    vec = pl.BlockSpec((1, D), lambda i: (0, 0))
    return pl.pallas_call(
        _ffn_body,
        grid=(N // BLK,),
        in_specs=[row, mat, vec, mat, vec, vec, vec],
        out_specs=row,
        out_shape=jax.ShapeDtypeStruct((N, D), jnp.float32),
    )(xa, w1, b1.reshape(1, D), w2, b2.reshape(1, D),
      g.reshape(1, D), b.reshape(1, D))


def _sc_deg_body(dst_hbm, deg_hbm, acc, rows, didx, sem):
    c = lax.axis_index("c")
    s = lax.axis_index("s")

    @pl.loop(0, CHUNK)
    def _(r):
        @pl.loop(0, D, step=16)
        def _(k):
            rows[r, pl.ds(k, 16)] = jnp.zeros((16,), jnp.float32)

    base = s * SUB_ROWS

    @pl.loop(0, SUB_ROWS // ZBLK)
    def _(k):
        pltpu.sync_copy(rows.at[pl.ds(0, ZBLK)],
                        acc.at[pl.ds(base + k * ZBLK, ZBLK)])

    @pl.when(s == NS - 1)
    def _():
        pltpu.sync_copy(rows.at[pl.ds(0, TAIL)],
                        acc.at[pl.ds(NS * SUB_ROWS, TAIL)])

    @pl.loop(0, CHUNK)
    def _(r):
        @pl.loop(0, D, step=16)
        def _(k):
            rows[r, pl.ds(k, 16)] = jnp.ones((16,), jnp.float32)

    w = c * NS + s
    pltpu.sync_copy(dst_hbm.at[pl.ds(w * WCH, WCH)], didx)

    plsc.subcore_barrier()

    @pl.loop(0, WCH)
    def _(k):
        pltpu.sync_copy(rows, acc.at[didx.at[k]], add=True)

    plsc.subcore_barrier()

    obase = c * N + s * SUB_ROWS

    @pl.loop(0, SUB_ROWS // ZBLK)
    def _(k):
        pltpu.sync_copy(acc.at[pl.ds(base + k * ZBLK, ZBLK)],
                        deg_hbm.at[pl.ds(obase + k * ZBLK, ZBLK)])

    @pl.when(s == NS - 1)
    def _():
        pltpu.sync_copy(acc.at[pl.ds(NS * SUB_ROWS, TAIL)],
                        deg_hbm.at[pl.ds(c * N + NS * SUB_ROWS, TAIL)])


@functools.cache
def _get_sc_deg():
  return pl.kernel(
    _sc_deg_body,
    out_type=jax.ShapeDtypeStruct((NC * N, D), jnp.float32),
    mesh=plsc.VectorSubcoreMesh(core_axis_name="c", subcore_axis_name="s",
                                num_cores=NC, num_subcores=NS),
    scratch_types=[
        pltpu.VMEM_SHARED((NA, D), jnp.float32),
        pltpu.VMEM((CHUNK, D), jnp.float32),
        pltpu.VMEM((WCH, CHUNK), jnp.int32),
        pltpu.SemaphoreType.DMA,
    ],
  )


def _sc_body(ff_hbm, src_hbm, dst_hbm, out_hbm,
             acc, rows0, rows1, sidx, didx, sem0, sem1):
    c = lax.axis_index("c")
    s = lax.axis_index("s")

    @pl.loop(0, CHUNK)
    def _(r):
        @pl.loop(0, D, step=16)
        def _(k):
            rows0[r, pl.ds(k, 16)] = jnp.zeros((16,), jnp.float32)

    base = s * SUB_ROWS

    @pl.loop(0, SUB_ROWS // ZBLK)
    def _(k):
        pltpu.sync_copy(rows0.at[pl.ds(0, ZBLK)],
                        acc.at[pl.ds(base + k * ZBLK, ZBLK)])

    @pl.when(s == NS - 1)
    def _():
        pltpu.sync_copy(rows0.at[pl.ds(0, TAIL)],
                        acc.at[pl.ds(NS * SUB_ROWS, TAIL)])

    w = c * NS + s

    plsc.subcore_barrier()

    @pl.loop(0, 2)
    def _(h):
        hb = w * WCH + h * HW
        pltpu.sync_copy(src_hbm.at[pl.ds(hb, HW)], sidx)
        pltpu.sync_copy(dst_hbm.at[pl.ds(hb, HW)], didx)

        pltpu.async_copy(ff_hbm.at[sidx.at[0]], rows0, sem0)

        @pl.loop(0, HW // 2)
        def _(p):
            k0 = 2 * p
            pltpu.make_async_copy(ff_hbm.at[sidx.at[0]], rows0, sem0).wait()
            pltpu.async_copy(ff_hbm.at[sidx.at[k0 + 1]], rows1, sem1)
            pltpu.sync_copy(rows0, acc.at[didx.at[k0]], add=True)
            pltpu.make_async_copy(ff_hbm.at[sidx.at[0]], rows1, sem1).wait()

            @pl.when(k0 + 2 < HW)
            def _():
                pltpu.async_copy(ff_hbm.at[sidx.at[k0 + 2]], rows0, sem0)

            pltpu.sync_copy(rows1, acc.at[didx.at[k0 + 1]], add=True)

    plsc.subcore_barrier()

    obase = c * N + s * SUB_ROWS

    @pl.loop(0, SUB_ROWS // ZBLK)
    def _(k):
        pltpu.sync_copy(acc.at[pl.ds(base + k * ZBLK, ZBLK)],
                        out_hbm.at[pl.ds(obase + k * ZBLK, ZBLK)])

    @pl.when(s == NS - 1)
    def _():
        pltpu.sync_copy(acc.at[pl.ds(NS * SUB_ROWS, TAIL)],
                        out_hbm.at[pl.ds(c * N + NS * SUB_ROWS, TAIL)])


@functools.cache
def _get_sc_seg():
  return pl.kernel(
    _sc_body,
    out_type=jax.ShapeDtypeStruct((NC * N, D), jnp.float32),
    mesh=plsc.VectorSubcoreMesh(core_axis_name="c", subcore_axis_name="s",
                                num_cores=NC, num_subcores=NS),
    scratch_types=[
        pltpu.VMEM_SHARED((NA, D), jnp.float32),
        pltpu.VMEM((CHUNK, D), jnp.float32),
        pltpu.VMEM((CHUNK, D), jnp.float32),
        pltpu.VMEM((HW, CHUNK), jnp.int32),
        pltpu.VMEM((HW, CHUNK), jnp.int32),
        pltpu.SemaphoreType.DMA,
        pltpu.SemaphoreType.DMA,
    ],
  )


def _agg_body(p_ref, deg_ref, ff_ref, wg_ref, bg_ref, g_ref, b_ref, o_ref):
    summ = p_ref[0] + p_ref[1]
    deg = deg_ref[0, :, 0:1] + deg_ref[1, :, 0:1]
    agg = summ / jnp.maximum(deg, 1.0)
    h = jnp.maximum(
        jnp.dot(agg, wg_ref[...], preferred_element_type=jnp.float32)
        + bg_ref[...], 0.0)
    o_ref[...] = _ln(h + ff_ref[...], g_ref[...], b_ref[...])


def _agg_ln(part, deg, ff, wg, bg, g, b):
    return pl.pallas_call(
        _agg_body,
        grid=(N // BLK,),
        in_specs=[
            pl.BlockSpec((2, BLK, D), lambda i: (0, i, 0)),
            pl.BlockSpec((2, BLK, D), lambda i: (0, i, 0)),
            pl.BlockSpec((BLK, D), lambda i: (i, 0)),
            pl.BlockSpec((D, D), lambda i: (0, 0)),
            pl.BlockSpec((1, D), lambda i: (0, 0)),
            pl.BlockSpec((1, D), lambda i: (0, 0)),
            pl.BlockSpec((1, D), lambda i: (0, 0)),
        ],
        out_specs=pl.BlockSpec((BLK, D), lambda i: (i, 0)),
        out_shape=jax.ShapeDtypeStruct((N, D), jnp.float32),
    )(part.reshape(NC, N, D), deg.reshape(NC, N, D), ff, wg,
      bg.reshape(1, D), g.reshape(1, D), b.reshape(1, D))


def kernel(x, edge_index,
           Wff1_0, bff1_0, Wff2_0, bff2_0, gff_0, boff_0, Wg_0, bg_0, gln_0, bln_0,
           Wff1_1, bff1_1, Wff2_1, bff2_1, gff_1, boff_1, Wg_1, bg_1, gln_1, bln_1):
    npad = NCHP * CHUNK - E
    pad_src = jnp.tile(jnp.arange(CHUNK, dtype=jnp.int32), npad // CHUNK)
    pad_dst = jnp.tile(jnp.arange(N, N + CHUNK, dtype=jnp.int32), npad // CHUNK)
    src = jnp.concatenate([edge_index[0], pad_src]).reshape(NCHP, CHUNK)
    dst = jnp.concatenate([edge_index[1], pad_dst]).reshape(NCHP, CHUNK)
    layers = [
        (Wff1_0, bff1_0, Wff2_0, bff2_0, gff_0, boff_0, Wg_0, bg_0, gln_0, bln_0),
        (Wff1_1, bff1_1, Wff2_1, bff2_1, gff_1, boff_1, Wg_1, bg_1, gln_1, bln_1),
    ]
    deg = _get_sc_deg()(dst)
    out = x
    for (w1, b1, w2, b2, gff, boff, wg, bg, gln, bln) in layers:
        ff = _ffn_ln(out, w1, b1, w2, b2, gff, boff)
        part = _get_sc_seg()(ff, src, dst)
        out = _agg_ln(part, deg, ff, wg, bg, gln, bln)
    return out
